# Initial kernel scaffold; baseline (speedup 1.0000x reference)
import jax
import jax.numpy as jnp
from jax.experimental import pallas as pl


def kernel(x, edge_index, W1, b1, W2, b2):
    raise NotImplementedError("write your pallas kernel here")


# speedup vs baseline: 391.6438x; 391.6438x over previous
import functools

import jax
import jax.numpy as jnp
from jax import lax
from jax.experimental import pallas as pl
from jax.experimental.pallas import tpu as pltpu
from jax.experimental.pallas import tpu_sc as plsc

N_NODES = 100000
ROWS_N = 782
N_PAD = ROWS_N * 128
NCHUNK = N_PAD // 16

E_EDGES = 6400000
NC, NS, LANES = 2, 16, 16
NW = NC * NS
ROW = 128
R_TILE = 1568
E_PAD = NW * R_TILE * ROW
EROWS = E_PAD // ROW
SB = 16
NB = R_TILE // SB


def _mesh():
    return plsc.VectorSubcoreMesh(
        core_axis_name="c", subcore_axis_name="s", num_cores=NC, num_subcores=NS
    )


def _zero_acc(zbuf, acc, sid):
    def zf(j, c):
        zbuf[pl.ds(j * 16, 16)] = jnp.zeros((16,), jnp.float32)
        return c

    lax.fori_loop(0, NCHUNK // 16, zf, 0)
    pltpu.sync_copy(zbuf, acc.at[pl.ds(sid * NCHUNK, NCHUNK)])


def _writeout(zbuf, acc, out_hbm, cid, sid):
    pltpu.sync_copy(acc.at[pl.ds(sid * NCHUNK, NCHUNK)], zbuf)
    pltpu.sync_copy(zbuf, out_hbm.at[pl.ds(cid * N_PAD + sid * NCHUNK, NCHUNK)])


def _deg_body(dst_hbm, out_hbm, stage, ones_v, zbuf, acc, sem_st, sem_sc):
    cid = lax.axis_index("c")
    sid = lax.axis_index("s")
    wid = cid * NS + sid
    _zero_acc(zbuf, acc, sid)
    for k in range(ROW // LANES):
        ones_v[pl.ds(k * 16, 16)] = jnp.ones((16,), jnp.float32)
    plsc.subcore_barrier()
    base = wid * R_TILE

    def start(b, slot):
        pltpu.async_copy(dst_hbm.at[pl.ds(base + b * SB, SB)], stage.at[slot], sem_st)

    def wait(slot):
        pltpu.make_async_copy(dst_hbm.at[pl.ds(0, SB)], stage.at[slot], sem_st).wait()

    def scat(slot):
        descs = [
            pltpu.async_copy(ones_v, acc.at[stage.at[slot, j]], sem_sc, add=True)
            for j in range(SB)
        ]
        for d in descs:
            d.wait()

    start(0, 0)

    def body(p, c):
        b0 = 2 * p
        start(b0 + 1, 1)
        wait(0)
        scat(0)

        @pl.when(p + 1 < NB // 2)
        def _():
            start(b0 + 2, 0)

        wait(1)
        scat(1)
        return c

    lax.fori_loop(0, NB // 2, body, 0)
    plsc.subcore_barrier()
    _writeout(zbuf, acc, out_hbm, cid, sid)


_deg_call = functools.partial(
    pl.kernel,
    out_type=jax.ShapeDtypeStruct((NC * N_PAD,), jnp.float32),
    mesh=_mesh(),
    compiler_params=pltpu.CompilerParams(needs_layout_passes=False),
    scratch_types=[
        pltpu.VMEM((2, SB, ROW), jnp.int32),
        pltpu.VMEM((ROW,), jnp.float32),
        pltpu.VMEM((NCHUNK,), jnp.float32),
        pltpu.VMEM_SHARED((N_PAD,), jnp.float32),
        pltpu.SemaphoreType.DMA,
        pltpu.SemaphoreType.DMA,
    ],
)(_deg_body)


def _agg_body(
    src_hbm, dst_hbm, tab_hbm, out_hbm,
    stage_s, stage_d, vals, table_v, zbuf, acc, sem_st, sem_sc,
):
    cid = lax.axis_index("c")
    sid = lax.axis_index("s")
    wid = cid * NS + sid
    _zero_acc(zbuf, acc, sid)
    pltpu.sync_copy(tab_hbm, table_v)
    plsc.subcore_barrier()
    base = wid * R_TILE

    def start(b, slot):
        pltpu.async_copy(src_hbm.at[pl.ds(base + b * SB, SB)], stage_s.at[slot], sem_st)
        pltpu.async_copy(dst_hbm.at[pl.ds(base + b * SB, SB)], stage_d.at[slot], sem_st)

    def wait(slot):
        pltpu.make_async_copy(src_hbm.at[pl.ds(0, SB)], stage_s.at[slot], sem_st).wait()
        pltpu.make_async_copy(dst_hbm.at[pl.ds(0, SB)], stage_d.at[slot], sem_st).wait()

    def proc(slot):
        for j in range(SB):
            for k in range(ROW // LANES):
                idx = stage_s[slot, j, pl.ds(k * 16, 16)]
                vals[slot, j, pl.ds(k * 16, 16)] = plsc.load_gather(table_v, [idx])
        descs = [
            pltpu.async_copy(
                vals.at[slot, j], acc.at[stage_d.at[slot, j]], sem_sc, add=True
            )
            for j in range(SB)
        ]
        for d in descs:
            d.wait()

    start(0, 0)

    def body(p, c):
        b0 = 2 * p
        start(b0 + 1, 1)
        wait(0)
        proc(0)

        @pl.when(p + 1 < NB // 2)
        def _():
            start(b0 + 2, 0)

        wait(1)
        proc(1)
        return c

    lax.fori_loop(0, NB // 2, body, 0)
    plsc.subcore_barrier()
    _writeout(zbuf, acc, out_hbm, cid, sid)


_agg_call = functools.partial(
    pl.kernel,
    out_type=jax.ShapeDtypeStruct((NC * N_PAD,), jnp.float32),
    mesh=_mesh(),
    compiler_params=pltpu.CompilerParams(needs_layout_passes=False),
    scratch_types=[
        pltpu.VMEM((2, SB, ROW), jnp.int32),
        pltpu.VMEM((2, SB, ROW), jnp.int32),
        pltpu.VMEM((2, SB, ROW), jnp.float32),
        pltpu.VMEM((N_PAD,), jnp.float32),
        pltpu.VMEM((NCHUNK,), jnp.float32),
        pltpu.VMEM_SHARED((N_PAD,), jnp.float32),
        pltpu.SemaphoreType.DMA,
        pltpu.SemaphoreType.DMA,
    ],
)(_agg_body)


def _ew1_body(degp, xp, dinv_o, g1_o):
    deg = degp[0] + degp[1] + 1.0
    dinv = lax.rsqrt(deg)
    dinv_o[...] = dinv
    g1_o[...] = dinv * xp[...]


def _ew2_body(rawp, dinv, xp, w1, b1, w2, h2_o, g2_o):
    dv = dinv[...]
    agg1 = dv * (rawp[0] + rawp[1]) + dv * dv * xp[...]
    acc = jnp.zeros_like(agg1)
    for j in range(8):
        acc = acc + jnp.maximum(agg1 * w1[0, j] + b1[j], 0.0) * w2[j, 0]
    h2_o[...] = acc
    g2_o[...] = dv * acc


def _ew3_body(rawp, dinv, h2, b2, out_o):
    dv = dinv[...]
    out_o[...] = dv * (rawp[0] + rawp[1]) + dv * dv * h2[...] + b2[0]


_SMEM = pl.BlockSpec(memory_space=pltpu.SMEM)


def _ew1(degp, xp):
    return pl.pallas_call(
        _ew1_body,
        out_shape=[
            jax.ShapeDtypeStruct((ROWS_N, 128), jnp.float32),
            jax.ShapeDtypeStruct((ROWS_N, 128), jnp.float32),
        ],
    )(degp, xp)


def _ew2(rawp, dinv, xp, w1, b1, w2):
    return pl.pallas_call(
        _ew2_body,
        in_specs=[pl.BlockSpec(), pl.BlockSpec(), pl.BlockSpec(), _SMEM, _SMEM, _SMEM],
        out_shape=[
            jax.ShapeDtypeStruct((ROWS_N, 128), jnp.float32),
            jax.ShapeDtypeStruct((ROWS_N, 128), jnp.float32),
        ],
    )(rawp, dinv, xp, w1, b1, w2)


def _ew3(rawp, dinv, h2, b2):
    return pl.pallas_call(
        _ew3_body,
        in_specs=[pl.BlockSpec(), pl.BlockSpec(), pl.BlockSpec(), _SMEM],
        out_shape=jax.ShapeDtypeStruct((ROWS_N, 128), jnp.float32),
    )(rawp, dinv, h2, b2)


def kernel(x, edge_index, W1, b1, W2, b2):
    xf = x[:, 0]
    xp = jnp.pad(xf, (0, N_PAD - N_NODES)).reshape(ROWS_N, 128)
    pad = jnp.full((2, E_PAD - E_EDGES), N_NODES, dtype=edge_index.dtype)
    ei = jnp.concatenate([edge_index, pad], axis=1)
    srcp = ei[0].reshape(EROWS, ROW)
    dstp = ei[1].reshape(EROWS, ROW)

    degp = _deg_call(dstp)
    dinv, g1 = _ew1(degp.reshape(NC, ROWS_N, 128), xp)
    raw1p = _agg_call(srcp, dstp, g1.reshape(N_PAD))
    h2, g2 = _ew2(raw1p.reshape(NC, ROWS_N, 128), dinv, xp, W1, b1, W2)
    raw2p = _agg_call(srcp, dstp, g2.reshape(N_PAD))
    out = _ew3(raw2p.reshape(NC, ROWS_N, 128), dinv, h2, b2)
    return out.reshape(N_PAD)[:N_NODES][:, None]
